# Initial kernel scaffold; baseline (speedup 1.0000x reference)
import jax
import jax.numpy as jnp
from jax import lax
from jax.experimental import pallas as pl
from jax.experimental.pallas import tpu as pltpu
from jax.experimental.pallas import tpu_sc as plsc

KMER = 1024
B, L = 4096, 200
TOT = B * L
NW = 32
PER_TILE = TOT // NW
CH = 12800
NCHUNK = PER_TILE // CH


def _sc_body(idx_hbm, mask_hbm, wt_hbm, r_hbm, s_hbm,
             rates_hbm, csp_hbm,
             idx_v, mask_v, wt_v, rates_v, r_tab, s_tab):
    wid = lax.axis_index("s") * 2 + lax.axis_index("c")
    base = wid * PER_TILE
    pltpu.sync_copy(r_hbm, r_tab)
    pltpu.sync_copy(s_hbm, s_tab)
    lanes4 = lax.iota(jnp.int32, (16,)) * 4

    for chunk in range(NCHUNK):
        off = base + chunk * CH
        pltpu.sync_copy(idx_hbm.at[pl.ds(off, CH)], idx_v)
        pltpu.sync_copy(mask_hbm.at[pl.ds(off, CH)], mask_v)
        pltpu.sync_copy(wt_hbm.at[pl.ds(off * 4, CH * 4)], wt_v)

        def body(j, carry):
            idx = idx_v[pl.ds(j * 16, 16)]
            m = mask_v[pl.ds(j * 16, 16)]
            r = plsc.load_gather(r_tab, [idx])
            rates_v[pl.ds(j * 16, 16)] = jnp.exp(r * m)
            idx4 = idx * 4
            tgt = lanes4 + j * 64
            for c in range(4):
                s_c = plsc.load_gather(s_tab, [idx4 + c])
                plsc.addupdate_scatter(wt_v, [tgt + c], s_c * m)
            return carry

        lax.fori_loop(0, CH // 16, body, 0)

        pltpu.sync_copy(rates_v, rates_hbm.at[pl.ds(off, CH)])
        pltpu.sync_copy(wt_v, csp_hbm.at[pl.ds(off * 4, CH * 4)])


@jax.jit
def _run(idx_flat, mask_flat, wt_flat, r_flat, s_flat):
    mesh = plsc.VectorSubcoreMesh(core_axis_name="c", subcore_axis_name="s")
    return pl.kernel(
        _sc_body,
        out_type=[jax.ShapeDtypeStruct((TOT,), jnp.float32),
                  jax.ShapeDtypeStruct((TOT * 4,), jnp.float32)],
        mesh=mesh,
        scratch_types=[
            pltpu.VMEM((CH,), jnp.int32),
            pltpu.VMEM((CH,), jnp.float32),
            pltpu.VMEM((CH * 4,), jnp.float32),
            pltpu.VMEM((CH,), jnp.float32),
            pltpu.VMEM((KMER,), jnp.float32),
            pltpu.VMEM((KMER * 4,), jnp.float32),
        ],
    )(idx_flat, mask_flat, wt_flat, r_flat, s_flat)


def kernel(encoded_parents, masks, wt_base_modifier, r_table, s_table):
    idx_flat = encoded_parents.reshape(-1).astype(jnp.int32)
    mask_flat = masks.reshape(-1)
    wt_flat = wt_base_modifier.reshape(-1)
    rates, csp = _run(idx_flat, mask_flat, wt_flat,
                      r_table.reshape(-1), s_table.reshape(-1))
    return rates.reshape(B, L), csp.reshape(B, L, 4)


# speedup vs baseline: 6.1574x; 6.1574x over previous
import jax
import jax.numpy as jnp
from jax import lax
from jax.experimental import pallas as pl
from jax.experimental.pallas import tpu as pltpu
from jax.experimental.pallas import tpu_sc as plsc

KMER = 1024
B, L = 4096, 200
TOT = B * L
NW = 32
PER_TILE = TOT // NW
CH = 12800
NCHUNK = PER_TILE // CH


def _sc_body(idx_hbm, mask_hbm, wt_hbm, r_hbm, s_hbm,
             rates_hbm, csp_hbm,
             idx_v, mask_v, wt_v, rates_v, r_tab, s_tab):
    wid = lax.axis_index("s") * 2 + lax.axis_index("c")
    base = wid * PER_TILE
    pltpu.sync_copy(r_hbm, r_tab)
    pltpu.sync_copy(s_hbm, s_tab)
    lanes4 = lax.iota(jnp.int32, 16) * 4

    for chunk in range(NCHUNK):
        off = base + chunk * CH
        pltpu.sync_copy(idx_hbm.at[pl.ds(off, CH)], idx_v)
        pltpu.sync_copy(mask_hbm.at[pl.ds(off, CH)], mask_v)
        pltpu.sync_copy(wt_hbm.at[pl.ds(off * 4, CH * 4)], wt_v)

        def body(j, carry):
            idx = idx_v[pl.ds(j * 16, 16)]
            m = mask_v[pl.ds(j * 16, 16)]
            r = plsc.load_gather(r_tab, [idx])
            rates_v[pl.ds(j * 16, 16)] = jnp.exp(r * m)
            idx4 = idx * 4
            tgt = lanes4 + j * 64
            for c in range(4):
                s_c = plsc.load_gather(s_tab, [idx4 + c])
                plsc.addupdate_scatter(wt_v, [tgt + c], s_c * m)
            return carry

        lax.fori_loop(0, CH // 16, body, 0)

        pltpu.sync_copy(rates_v, rates_hbm.at[pl.ds(off, CH)])
        pltpu.sync_copy(wt_v, csp_hbm.at[pl.ds(off * 4, CH * 4)])


@jax.jit
def _run(idx_flat, mask_flat, wt_flat, r_flat, s_flat):
    mesh = plsc.VectorSubcoreMesh(core_axis_name="c", subcore_axis_name="s")
    return pl.kernel(
        _sc_body,
        out_type=[jax.ShapeDtypeStruct((TOT,), jnp.float32),
                  jax.ShapeDtypeStruct((TOT * 4,), jnp.float32)],
        mesh=mesh,
        compiler_params=pltpu.CompilerParams(needs_layout_passes=False),
        scratch_types=[
            pltpu.VMEM((CH,), jnp.int32),
            pltpu.VMEM((CH,), jnp.float32),
            pltpu.VMEM((CH * 4,), jnp.float32),
            pltpu.VMEM((CH,), jnp.float32),
            pltpu.VMEM((KMER,), jnp.float32),
            pltpu.VMEM((KMER * 4,), jnp.float32),
        ],
    )(idx_flat, mask_flat, wt_flat, r_flat, s_flat)


def kernel(encoded_parents, masks, wt_base_modifier, r_table, s_table):
    idx_flat = encoded_parents.reshape(-1).astype(jnp.int32)
    mask_flat = masks.reshape(-1)
    wt_flat = wt_base_modifier.reshape(-1)
    rates, csp = _run(idx_flat, mask_flat, wt_flat,
                      r_table.reshape(-1), s_table.reshape(-1))
    return rates.reshape(B, L), csp.reshape(B, L, 4)


# speedup vs baseline: 75.1181x; 12.1996x over previous
import jax
import jax.numpy as jnp
from jax import lax
from jax.experimental import pallas as pl
from jax.experimental.pallas import tpu as pltpu
from jax.experimental.pallas import tpu_sc as plsc

KMER = 1024
B, L = 4096, 200
NW = 32
QB = B // NW
CHL = 40
NCHL = L // CHL


def _sc_body(idx_hbm, mask_hbm, w4_hbm, r_hbm, s_hbm,
             rates_hbm, csp_hbm,
             idx_v, mask_v, wt_v, rates_v, r_tab, s_tab):
    bt = lax.axis_index("s") * 2 + lax.axis_index("c")
    pltpu.sync_copy(r_hbm, r_tab)
    pltpu.sync_copy(s_hbm, s_tab)
    pltpu.sync_copy(idx_hbm.at[pl.ds(bt * QB, QB), :], idx_v)
    pltpu.sync_copy(mask_hbm.at[pl.ds(bt * QB, QB), :], mask_v)

    iota = lax.iota(jnp.int32, 16)
    rows = [iota + 16 * k for k in range(QB // 16)]

    for chunk in range(NCHL):
        l0 = chunk * CHL
        pltpu.sync_copy(w4_hbm.at[pl.ds(l0, CHL), bt], wt_v)

        def body(l_loc, carry):
            l = l0 + l_loc
            lt = l_loc // 8
            s = l_loc % 8
            lvec = jnp.full((16,), l, jnp.int32)
            for k in range(QB // 16):
                idx = plsc.load_gather(idx_v, [rows[k], lvec])
                m = plsc.load_gather(mask_v, [rows[k], lvec])
                r = plsc.load_gather(r_tab, [idx])
                rates_v[lt, s, pl.ds(16 * k, 16)] = jnp.exp(r * m)
                idx4 = idx * 4
                for c in range(4):
                    s_c = plsc.load_gather(s_tab, [idx4 + c])
                    wt_v[l_loc, c, pl.ds(16 * k, 16)] = (
                        wt_v[l_loc, c, pl.ds(16 * k, 16)] + s_c * m)
            return carry

        lax.fori_loop(0, CHL, body, 0)

        pltpu.sync_copy(wt_v, csp_hbm.at[pl.ds(l0, CHL), bt])
        pltpu.sync_copy(rates_v, rates_hbm.at[pl.ds(l0 // 8, CHL // 8), bt])


@jax.jit
def _run(idx2, mask2, w4, r_flat, s_flat):
    mesh = plsc.VectorSubcoreMesh(core_axis_name="c", subcore_axis_name="s")
    return pl.kernel(
        _sc_body,
        out_type=[jax.ShapeDtypeStruct((L // 8, NW, 8, QB), jnp.float32),
                  jax.ShapeDtypeStruct((L, NW, 4, QB), jnp.float32)],
        mesh=mesh,
        compiler_params=pltpu.CompilerParams(needs_layout_passes=False),
        scratch_types=[
            pltpu.VMEM((QB, L), jnp.int32),
            pltpu.VMEM((QB, L), jnp.float32),
            pltpu.VMEM((CHL, 4, QB), jnp.float32),
            pltpu.VMEM((CHL // 8, 8, QB), jnp.float32),
            pltpu.VMEM((KMER,), jnp.float32),
            pltpu.VMEM((KMER * 4,), jnp.float32),
        ],
    )(idx2, mask2, w4, r_flat, s_flat)


def kernel(encoded_parents, masks, wt_base_modifier, r_table, s_table):
    idx2 = encoded_parents.astype(jnp.int32)
    w4 = wt_base_modifier.reshape(NW, QB, L, 4).transpose(2, 0, 3, 1)
    rates5, csp4 = _run(idx2, masks, w4,
                        r_table.reshape(-1), s_table.reshape(-1))
    rates = rates5.transpose(0, 2, 1, 3).reshape(L, B).T
    csp = csp4.transpose(1, 3, 0, 2).reshape(B, L, 4)
    return rates, csp


# speedup vs baseline: 128.4773x; 1.7103x over previous
import jax
import jax.numpy as jnp
from jax import lax
from jax.experimental import pallas as pl
from jax.experimental.pallas import tpu as pltpu
from jax.experimental.pallas import tpu_sc as plsc

KMER = 1024
B, L = 4096, 200
NW = 32
QB = B // NW
CHL = 40
NCHL = L // CHL


def _sc_body(idx_hbm, mask_hbm, w4_hbm, r_hbm, s_hbm,
             rates_hbm, csp_hbm,
             idx_v, mask_v, wt_v0, wt_v1, rates_v0, rates_v1,
             r_tab, s_tab,
             sin0, sin1, scsp0, scsp1, srat0, srat1):
    bt = lax.axis_index("s") * 2 + lax.axis_index("c")
    pltpu.sync_copy(r_hbm, r_tab)
    pltpu.sync_copy(s_hbm, s_tab)
    pltpu.sync_copy(idx_hbm.at[pl.ds(bt * QB, QB), :], idx_v)
    pltpu.sync_copy(mask_hbm.at[pl.ds(bt * QB, QB), :], mask_v)

    wts = [wt_v0, wt_v1]
    rvs = [rates_v0, rates_v1]
    sins = [sin0, sin1]
    scsps = [scsp0, scsp1]
    srats = [srat0, srat1]

    iota = lax.iota(jnp.int32, 16)
    rows = [iota + 16 * k for k in range(QB // 16)]

    def in_copy(c, b):
        return pltpu.async_copy(
            w4_hbm.at[pl.ds(c * CHL, CHL), bt], wts[b], sins[b])

    def out_copies(c, b):
        return (pltpu.async_copy(
                    wts[b], csp_hbm.at[pl.ds(c * CHL, CHL), bt], scsps[b]),
                pltpu.async_copy(
                    rvs[b], rates_hbm.at[pl.ds(c * CHL // 8, CHL // 8), bt],
                    srats[b]))

    in_h = {0: in_copy(0, 0)}
    out_h = {}
    for c in range(NCHL):
        b = c % 2
        if c + 1 < NCHL:
            if c >= 1:
                for h in out_h.pop(c - 1):
                    h.wait()
            in_h[c + 1] = in_copy(c + 1, 1 - b)
        in_h.pop(c).wait()

        wt_v = wts[b]
        rates_v = rvs[b]

        @plsc.parallel_loop(0, CHL, unroll=2)
        def body(l_loc):
            lt = l_loc >> 3
            s = l_loc & 7
            lvec = jnp.full((16,), l_loc + c * CHL, jnp.int32)
            for k in range(QB // 16):
                idx = plsc.load_gather(idx_v, [rows[k], lvec])
                m = plsc.load_gather(mask_v, [rows[k], lvec])
                r = plsc.load_gather(r_tab, [idx])
                rates_v[lt, s, pl.ds(16 * k, 16)] = jnp.exp(r * m)
                idx4 = idx * 4
                for cc in range(4):
                    s_c = plsc.load_gather(s_tab, [idx4 + cc])
                    wt_v[l_loc, cc, pl.ds(16 * k, 16)] = (
                        wt_v[l_loc, cc, pl.ds(16 * k, 16)] + s_c * m)

        out_h[c] = out_copies(c, b)

    for c in (NCHL - 2, NCHL - 1):
        for h in out_h.pop(c, ()):
            h.wait()


@jax.jit
def _run(idx2, mask2, w4, r_flat, s_flat):
    mesh = plsc.VectorSubcoreMesh(core_axis_name="c", subcore_axis_name="s")
    return pl.kernel(
        _sc_body,
        out_type=[jax.ShapeDtypeStruct((L // 8, NW, 8, QB), jnp.float32),
                  jax.ShapeDtypeStruct((L, NW, 4, QB), jnp.float32)],
        mesh=mesh,
        compiler_params=pltpu.CompilerParams(needs_layout_passes=False),
        scratch_types=[
            pltpu.VMEM((QB, L), jnp.int32),
            pltpu.VMEM((QB, L), jnp.float32),
            pltpu.VMEM((CHL, 4, QB), jnp.float32),
            pltpu.VMEM((CHL, 4, QB), jnp.float32),
            pltpu.VMEM((CHL // 8, 8, QB), jnp.float32),
            pltpu.VMEM((CHL // 8, 8, QB), jnp.float32),
            pltpu.VMEM((KMER,), jnp.float32),
            pltpu.VMEM((KMER * 4,), jnp.float32),
            pltpu.SemaphoreType.DMA,
            pltpu.SemaphoreType.DMA,
            pltpu.SemaphoreType.DMA,
            pltpu.SemaphoreType.DMA,
            pltpu.SemaphoreType.DMA,
            pltpu.SemaphoreType.DMA,
        ],
    )(idx2, mask2, w4, r_flat, s_flat)


def kernel(encoded_parents, masks, wt_base_modifier, r_table, s_table):
    idx2 = encoded_parents.astype(jnp.int32)
    w4 = wt_base_modifier.reshape(NW, QB, L, 4).transpose(2, 0, 3, 1)
    rates5, csp4 = _run(idx2, masks, w4,
                        r_table.reshape(-1), s_table.reshape(-1))
    rates = rates5.transpose(0, 2, 1, 3).reshape(L, B).T
    csp = csp4.transpose(1, 3, 0, 2).reshape(B, L, 4)
    return rates, csp


# speedup vs baseline: 134.2685x; 1.0451x over previous
import jax
import jax.numpy as jnp
from jax import lax
from jax.experimental import pallas as pl
from jax.experimental.pallas import tpu as pltpu
from jax.experimental.pallas import tpu_sc as plsc

KMER = 1024
B, L = 4096, 200
NW = 32
QB = B // NW
CHL = 40
NCHL = L // CHL


def _sc_body(idx_hbm, mask_hbm, w4_hbm, r_hbm, s_hbm,
             rates_hbm, csp_hbm,
             idx_v, mask_v, wt_v0, wt_v1, rates_v0, rates_v1,
             r_tab, s_tab,
             sin0, sin1, scsp0, scsp1, srat0, srat1):
    bt = lax.axis_index("s") * 2 + lax.axis_index("c")
    pltpu.sync_copy(r_hbm, r_tab)
    pltpu.sync_copy(s_hbm, s_tab)
    pltpu.sync_copy(idx_hbm.at[pl.ds(bt * QB, QB), :], idx_v)
    pltpu.sync_copy(mask_hbm.at[pl.ds(bt * QB, QB), :], mask_v)

    wts = [wt_v0, wt_v1]
    rvs = [rates_v0, rates_v1]
    sins = [sin0, sin1]
    scsps = [scsp0, scsp1]
    srats = [srat0, srat1]

    iota = lax.iota(jnp.int32, 16)
    rows = [iota + 16 * k for k in range(QB // 16)]

    def in_copy(c, b):
        return pltpu.async_copy(
            w4_hbm.at[pl.ds(c * CHL, CHL), bt], wts[b], sins[b])

    def out_copies(c, b):
        return (pltpu.async_copy(
                    wts[b], csp_hbm.at[pl.ds(c * CHL, CHL), bt], scsps[b]),
                pltpu.async_copy(
                    rvs[b], rates_hbm.at[pl.ds(c * CHL // 8, CHL // 8), bt],
                    srats[b]))

    in_h = {0: in_copy(0, 0)}
    out_h = {}
    for c in range(NCHL):
        b = c % 2
        if c + 1 < NCHL:
            if c >= 1:
                for h in out_h.pop(c - 1):
                    h.wait()
            in_h[c + 1] = in_copy(c + 1, 1 - b)
        in_h.pop(c).wait()

        wt_v = wts[b]
        rates_v = rvs[b]

        @plsc.parallel_loop(0, CHL, unroll=2)
        def body(l_loc):
            lt = l_loc >> 3
            s = l_loc & 7
            lvec = jnp.full((16,), l_loc + c * CHL, jnp.int32)
            for k in range(QB // 16):
                idx = plsc.load_gather(idx_v, [rows[k], lvec])
                m = plsc.load_gather(mask_v, [rows[k], lvec])
                r = plsc.load_gather(r_tab, [idx])
                rates_v[lt, s, pl.ds(16 * k, 16)] = jnp.exp(r * m)
                idx4 = idx * 4
                for cc in range(4):
                    s_c = plsc.load_gather(s_tab, [idx4 + cc])
                    plsc.addupdate(
                        wt_v.at[l_loc, cc, pl.ds(16 * k, 16)], s_c * m)

        out_h[c] = out_copies(c, b)

    for c in (NCHL - 2, NCHL - 1):
        for h in out_h.pop(c, ()):
            h.wait()


@jax.jit
def _run(idx2, mask2, w4, r_flat, s_flat):
    mesh = plsc.VectorSubcoreMesh(core_axis_name="c", subcore_axis_name="s")
    return pl.kernel(
        _sc_body,
        out_type=[jax.ShapeDtypeStruct((L // 8, NW, 8, QB), jnp.float32),
                  jax.ShapeDtypeStruct((L, NW, 4, QB), jnp.float32)],
        mesh=mesh,
        compiler_params=pltpu.CompilerParams(needs_layout_passes=False),
        scratch_types=[
            pltpu.VMEM((QB, L), jnp.int32),
            pltpu.VMEM((QB, L), jnp.float32),
            pltpu.VMEM((CHL, 4, QB), jnp.float32),
            pltpu.VMEM((CHL, 4, QB), jnp.float32),
            pltpu.VMEM((CHL // 8, 8, QB), jnp.float32),
            pltpu.VMEM((CHL // 8, 8, QB), jnp.float32),
            pltpu.VMEM((KMER,), jnp.float32),
            pltpu.VMEM((KMER * 4,), jnp.float32),
            pltpu.SemaphoreType.DMA,
            pltpu.SemaphoreType.DMA,
            pltpu.SemaphoreType.DMA,
            pltpu.SemaphoreType.DMA,
            pltpu.SemaphoreType.DMA,
            pltpu.SemaphoreType.DMA,
        ],
    )(idx2, mask2, w4, r_flat, s_flat)


def kernel(encoded_parents, masks, wt_base_modifier, r_table, s_table):
    idx2 = encoded_parents.astype(jnp.int32)
    w4 = wt_base_modifier.reshape(NW, QB, L, 4).transpose(2, 0, 3, 1)
    rates5, csp4 = _run(idx2, masks, w4,
                        r_table.reshape(-1), s_table.reshape(-1))
    rates = rates5.transpose(0, 2, 1, 3).reshape(L, B).T
    csp = csp4.transpose(1, 3, 0, 2).reshape(B, L, 4)
    return rates, csp
